# baseline (speedup 1.0000x reference)
import numpy as np
import jax
import jax.numpy as jnp
from jax.experimental import pallas as pl
from jax.experimental.pallas import tpu as pltpu

H_UNITS_K = 128


def _pos_enc_table_np(seq, h_units):
    pos = np.arange(seq).astype(np.float64)[:, None]
    i = np.arange(h_units).astype(np.float64)[None, :]
    enc = pos / np.power(10000.0, 2.0 * i / float(h_units))
    enc = enc.astype(np.float32)
    enc[:, 0::2] = np.sin(enc[:, 0::2])
    enc[:, 1::2] = np.cos(enc[:, 1::2])
    return enc


def kernel(inputs):
    bs, seq = inputs.shape
    h = H_UNITS_K
    table = jnp.asarray(_pos_enc_table_np(seq, h))

    BB = 32
    NQ = 8
    assert bs % BB == 0
    nchunks = bs // BB

    def body(tab_ref, out_ref, buf, sems):
        buf[...] = jnp.broadcast_to(tab_ref[...][None], (BB, seq, h))
        copies = []
        for c in range(nchunks):
            cp = pltpu.make_async_copy(
                buf, out_ref.at[pl.ds(c * BB, BB)], sems.at[c % NQ]
            )
            cp.start()
            copies.append(cp)
        for cp in copies:
            cp.wait()

    out = pl.pallas_call(
        body,
        in_specs=[pl.BlockSpec(memory_space=pltpu.VMEM)],
        out_specs=pl.BlockSpec(memory_space=pl.ANY),
        out_shape=jax.ShapeDtypeStruct((bs, seq, h), jnp.float32),
        scratch_shapes=[
            pltpu.VMEM((BB, seq, h), jnp.float32),
            pltpu.SemaphoreType.DMA((NQ,)),
        ],
    )(table)
    return out


# speedup vs baseline: 1.0556x; 1.0556x over previous
import numpy as np
import jax
import jax.numpy as jnp
from jax.experimental import pallas as pl

H_UNITS_K = 128


def _pos_enc_table_np(seq, h_units):
    pos = np.arange(seq).astype(np.float64)[:, None]
    i = np.arange(h_units).astype(np.float64)[None, :]
    enc = pos / np.power(10000.0, 2.0 * i / float(h_units))
    enc = enc.astype(np.float32)
    enc[:, 0::2] = np.sin(enc[:, 0::2])
    enc[:, 1::2] = np.cos(enc[:, 1::2])
    return enc


def kernel(inputs):
    bs, seq = inputs.shape
    h = H_UNITS_K
    table = jnp.asarray(_pos_enc_table_np(seq, h))

    BB = 32
    assert bs % BB == 0

    def body(tab_ref, out_ref):
        out_ref[...] = jnp.broadcast_to(tab_ref[...][None], (BB, seq, h))

    out = pl.pallas_call(
        body,
        grid=(bs // BB,),
        in_specs=[pl.BlockSpec((seq, h), lambda i: (0, 0))],
        out_specs=pl.BlockSpec((BB, seq, h), lambda i: (i, 0, 0)),
        out_shape=jax.ShapeDtypeStruct((bs, seq, h), jnp.float32),
    )(table)
    return out
